# baseline (device time: 14373 ns/iter reference)
import jax
import jax.numpy as jnp
from jax import lax
from jax.experimental import pallas as pl
from jax.experimental.pallas import tpu as pltpu

N_DEV = 4
B, SQ, SKV = 2, 128, 128
D_MODEL = 512
HQ_LOCAL, DH = 4, 64


def kernel(x, Wq, K_ext, V_ext, Wo):
    KW = HQ_LOCAL * DH
    K_loc = K_ext.reshape(B, SKV, 16 * DH)
    V_loc = V_ext.reshape(B, SKV, 16 * DH)

    HALF = D_MODEL // 2

    def body(x_ref, wq_ref, k_ref, v_ref, wo_ref, out_ref,
             comm_ref, kv_ref, vv_ref, copy_sems, send_sems, recv_sems):
        my_pos = lax.axis_index("i")

        kcopy = pltpu.make_async_copy(
            k_ref.at[:, :, pl.ds(my_pos * KW, KW)], kv_ref, copy_sems.at[0])
        vcopy = pltpu.make_async_copy(
            v_ref.at[:, :, pl.ds(my_pos * KW, KW)], vv_ref, copy_sems.at[1])
        kcopy.start()
        vcopy.start()
        pa = my_pos ^ 1
        pb = 3 - my_pos

        barrier_sem = pltpu.get_barrier_semaphore()
        for nbr in (pa, pb):
            pl.semaphore_signal(
                barrier_sem, inc=1,
                device_id=(nbr,), device_id_type=pl.DeviceIdType.MESH,
            )
        pl.semaphore_wait(barrier_sem, 2)

        def xchg(src_slot, dst_slot, sem_idx, partner):
            return pltpu.make_async_remote_copy(
                src_ref=comm_ref.at[src_slot], dst_ref=comm_ref.at[dst_slot],
                send_sem=send_sems.at[sem_idx], recv_sem=recv_sems.at[sem_idx],
                device_id=(partner,), device_id_type=pl.DeviceIdType.MESH,
            )

        r1_partner = (pa, pb)
        r2_partner = (pb, pa)

        x2 = x_ref[...].reshape(B * SQ, D_MODEL)
        q2 = jnp.dot(x2, wq_ref[...], preferred_element_type=jnp.float32)

        ri = lax.broadcasted_iota(jnp.int32, (SQ, SKV), 0)
        ci = lax.broadcasted_iota(jnp.int32, (SQ, SKV), 1)
        mask = ((ri // 64) == (ci // 64)).astype(jnp.float32)

        kcopy.wait()
        vcopy.wait()

        r1 = {}
        for b in range(B):
            kb = kv_ref[b]
            vb = vv_ref[b]
            heads = []
            for h in range(HQ_LOCAL):
                qbh = q2[b * SQ:(b + 1) * SQ, h * DH:(h + 1) * DH]
                kbh = kb[:, h * DH:(h + 1) * DH]
                vbh = vb[:, h * DH:(h + 1) * DH]
                s = lax.dot_general(
                    qbh, kbh, (((1,), (1,)), ((), ())),
                    preferred_element_type=jnp.float32,
                ) * 0.125
                e = jnp.exp(s) * mask
                w = e / jnp.sum(e, axis=-1, keepdims=True)
                heads.append(jnp.dot(w, vbh, preferred_element_type=jnp.float32))
            ctx_b = jnp.concatenate(heads, axis=1)
            partial_b = jnp.dot(
                ctx_b, wo_ref[...], preferred_element_type=jnp.float32
            )
            for half in range(2):
                comm_ref[b * 2 + half] = (
                    partial_b[:, half * HALF:(half + 1) * HALF]
                    .astype(jnp.bfloat16))
                r1[b, half] = xchg(
                    b * 2 + half, 4 + b * 2 + half, b * 2 + half,
                    r1_partner[half])
                r1[b, half].start()
            out_ref[b] = partial_b

        r2 = {}
        for b in range(B):
            for half in range(2):
                q_idx = b * 2 + half
                r1[b, half].wait()
                acc = (out_ref[b, :, half * HALF:(half + 1) * HALF]
                       + comm_ref[4 + q_idx].astype(jnp.float32))
                comm_ref[8 + q_idx] = acc.astype(jnp.bfloat16)
                r2[b, half] = xchg(8 + q_idx, 12 + q_idx, 4 + q_idx,
                                   r2_partner[half])
                r2[b, half].start()
                out_ref[b, :, half * HALF:(half + 1) * HALF] = acc

        for b in range(B):
            for half in range(2):
                q_idx = b * 2 + half
                r2[b, half].wait()
                out_ref[b, :, half * HALF:(half + 1) * HALF] += (
                    comm_ref[12 + q_idx].astype(jnp.float32))

    return pl.pallas_call(
        body,
        out_shape=jax.ShapeDtypeStruct((B, SQ, D_MODEL), jnp.float32),
        in_specs=[
            pl.BlockSpec(memory_space=pltpu.VMEM),
            pl.BlockSpec(memory_space=pltpu.VMEM),
            pl.BlockSpec(memory_space=pltpu.MemorySpace.HBM),
            pl.BlockSpec(memory_space=pltpu.MemorySpace.HBM),
            pl.BlockSpec(memory_space=pltpu.VMEM),
        ],
        out_specs=pl.BlockSpec(memory_space=pltpu.VMEM),
        scratch_shapes=[
            pltpu.VMEM((16, SQ, HALF), jnp.bfloat16),
            pltpu.VMEM((B, SKV, KW), jnp.float32),
            pltpu.VMEM((B, SKV, KW), jnp.float32),
            pltpu.SemaphoreType.DMA((2,)),
            pltpu.SemaphoreType.DMA((8,)),
            pltpu.SemaphoreType.DMA((8,)),
        ],
        compiler_params=pltpu.CompilerParams(collective_id=0),
    )(x, Wq, K_loc, V_loc, Wo)


# device time: 14162 ns/iter; 1.0149x vs baseline; 1.0149x over previous
import jax
import jax.numpy as jnp
from jax import lax
from jax.experimental import pallas as pl
from jax.experimental.pallas import tpu as pltpu

N_DEV = 4
B, SQ, SKV = 2, 128, 128
D_MODEL = 512
HQ_LOCAL, DH = 4, 64


def kernel(x, Wq, K_ext, V_ext, Wo):
    KW = HQ_LOCAL * DH
    K_loc = K_ext.reshape(B, SKV, 16 * DH)
    V_loc = V_ext.reshape(B, SKV, 16 * DH)

    HALF = D_MODEL // 2

    def body(x_ref, wq_ref, k_ref, v_ref, wo_ref, out_ref,
             comm_ref, send_sems, recv_sems):
        my_pos = lax.axis_index("i")
        pa = my_pos ^ 1
        pb = 3 - my_pos

        barrier_sem = pltpu.get_barrier_semaphore()
        for nbr in (pa, pb):
            pl.semaphore_signal(
                barrier_sem, inc=1,
                device_id=(nbr,), device_id_type=pl.DeviceIdType.MESH,
            )
        pl.semaphore_wait(barrier_sem, 2)

        def xchg(src_slot, dst_slot, sem_idx, partner):
            return pltpu.make_async_remote_copy(
                src_ref=comm_ref.at[src_slot], dst_ref=comm_ref.at[dst_slot],
                send_sem=send_sems.at[sem_idx], recv_sem=recv_sems.at[sem_idx],
                device_id=(partner,), device_id_type=pl.DeviceIdType.MESH,
            )

        r1_partner = (pa, pb)
        r2_partner = (pb, pa)

        x2 = x_ref[...].reshape(B * SQ, D_MODEL)
        q2 = jnp.dot(x2, wq_ref[...], preferred_element_type=jnp.float32)

        ri = lax.broadcasted_iota(jnp.int32, (SQ, SKV), 0)
        ci = lax.broadcasted_iota(jnp.int32, (SQ, SKV), 1)
        mask = ((ri // 64) == (ci // 64)).astype(jnp.float32)

        r1 = {}
        for b in range(B):
            kb = k_ref[b, :, pl.ds(my_pos * KW, KW)]
            vb = v_ref[b, :, pl.ds(my_pos * KW, KW)]
            heads = []
            for h in range(HQ_LOCAL):
                qbh = q2[b * SQ:(b + 1) * SQ, h * DH:(h + 1) * DH]
                kbh = kb[:, h * DH:(h + 1) * DH]
                vbh = vb[:, h * DH:(h + 1) * DH]
                s = lax.dot_general(
                    qbh, kbh, (((1,), (1,)), ((), ())),
                    preferred_element_type=jnp.float32,
                ) * 0.125
                e = jnp.exp(s) * mask
                w = e / jnp.sum(e, axis=-1, keepdims=True)
                heads.append(jnp.dot(w, vbh, preferred_element_type=jnp.float32))
            ctx_b = jnp.concatenate(heads, axis=1)
            partial_b = jnp.dot(
                ctx_b, wo_ref[...], preferred_element_type=jnp.float32
            )
            for half in range(2):
                comm_ref[b * 2 + half] = (
                    partial_b[:, half * HALF:(half + 1) * HALF]
                    .astype(jnp.bfloat16))
                r1[b, half] = xchg(
                    b * 2 + half, 4 + b * 2 + half, b * 2 + half,
                    r1_partner[half])
                r1[b, half].start()
            out_ref[b] = partial_b

        r2 = {}
        for b in range(B):
            for half in range(2):
                q_idx = b * 2 + half
                r1[b, half].wait()
                comm_ref[8 + q_idx] = comm_ref[q_idx] + comm_ref[4 + q_idx]
                r2[b, half] = xchg(8 + q_idx, 12 + q_idx, 4 + q_idx,
                                   r2_partner[half])
                r2[b, half].start()

        for b in range(B):
            for half in range(2):
                q_idx = b * 2 + half
                out_ref[b, :, half * HALF:(half + 1) * HALF] += (
                    comm_ref[4 + q_idx].astype(jnp.float32))

        for b in range(B):
            for half in range(2):
                q_idx = b * 2 + half
                r2[b, half].wait()
                out_ref[b, :, half * HALF:(half + 1) * HALF] += (
                    comm_ref[12 + q_idx].astype(jnp.float32))

    return pl.pallas_call(
        body,
        out_shape=jax.ShapeDtypeStruct((B, SQ, D_MODEL), jnp.float32),
        in_specs=[pl.BlockSpec(memory_space=pltpu.VMEM)] * 5,
        out_specs=pl.BlockSpec(memory_space=pltpu.VMEM),
        scratch_shapes=[
            pltpu.VMEM((16, SQ, HALF), jnp.bfloat16),
            pltpu.SemaphoreType.DMA((8,)),
            pltpu.SemaphoreType.DMA((8,)),
        ],
        compiler_params=pltpu.CompilerParams(collective_id=0),
    )(x, Wq, K_loc, V_loc, Wo)


# device time: 13271 ns/iter; 1.0830x vs baseline; 1.0671x over previous
import jax
import jax.numpy as jnp
from jax import lax
from jax.experimental import pallas as pl
from jax.experimental.pallas import tpu as pltpu

N_DEV = 4
B, SQ, SKV = 2, 128, 128
D_MODEL = 512
HQ_LOCAL, DH = 4, 64


def kernel(x, Wq, K_ext, V_ext, Wo):
    KW = HQ_LOCAL * DH
    my = lax.axis_index("i")
    K_loc = lax.dynamic_slice_in_dim(
        K_ext, my * HQ_LOCAL, HQ_LOCAL, axis=2).reshape(B, SKV, KW)
    V_loc = lax.dynamic_slice_in_dim(
        V_ext, my * HQ_LOCAL, HQ_LOCAL, axis=2).reshape(B, SKV, KW)

    HALF = D_MODEL // 2

    def body(x_ref, wq_ref, k_ref, v_ref, wo_ref, out_ref,
             comm_ref, send_sems, recv_sems):
        my_pos = lax.axis_index("i")
        pa = my_pos ^ 1
        pb = 3 - my_pos

        barrier_sem = pltpu.get_barrier_semaphore()
        for nbr in (pa, pb):
            pl.semaphore_signal(
                barrier_sem, inc=1,
                device_id=(nbr,), device_id_type=pl.DeviceIdType.MESH,
            )

        def xchg(src_slot, dst_slot, sem_idx, partner):
            return pltpu.make_async_remote_copy(
                src_ref=comm_ref.at[src_slot], dst_ref=comm_ref.at[dst_slot],
                send_sem=send_sems.at[sem_idx], recv_sem=recv_sems.at[sem_idx],
                device_id=(partner,), device_id_type=pl.DeviceIdType.MESH,
            )

        r1_partner = (pa, pb)
        r2_partner = (pb, pa)

        x2 = x_ref[...].reshape(B * SQ, D_MODEL)
        q2 = jnp.dot(x2, wq_ref[...], preferred_element_type=jnp.float32)

        ri = lax.broadcasted_iota(jnp.int32, (SQ, SKV), 0)
        ci = lax.broadcasted_iota(jnp.int32, (SQ, SKV), 1)
        mask = ((ri // 64) == (ci // 64)).astype(jnp.float32)

        r1 = {}
        for b in range(B):
            kb = k_ref[b]
            vb = v_ref[b]
            heads = []
            for h in range(HQ_LOCAL):
                qbh = q2[b * SQ:(b + 1) * SQ, h * DH:(h + 1) * DH]
                kbh = kb[:, h * DH:(h + 1) * DH]
                vbh = vb[:, h * DH:(h + 1) * DH]
                s = lax.dot_general(
                    qbh, kbh, (((1,), (1,)), ((), ())),
                    preferred_element_type=jnp.float32,
                ) * 0.125
                e = jnp.exp(s) * mask
                w = e / jnp.sum(e, axis=-1, keepdims=True)
                heads.append(jnp.dot(w, vbh, preferred_element_type=jnp.float32))
            ctx_b = jnp.concatenate(heads, axis=1)
            partial_b = jnp.dot(
                ctx_b, wo_ref[...], preferred_element_type=jnp.float32
            )
            for half in range(2):
                comm_ref[b * 2 + half] = (
                    partial_b[:, half * HALF:(half + 1) * HALF]
                    .astype(jnp.bfloat16))
            if b == 0:
                pl.semaphore_wait(barrier_sem, 2)
            for half in range(2):
                r1[b, half] = xchg(
                    b * 2 + half, 4 + b * 2 + half, b * 2 + half,
                    r1_partner[half])
                r1[b, half].start()
            out_ref[b] = partial_b

        r2 = {}
        for b in range(B):
            for half in range(2):
                q_idx = b * 2 + half
                r1[b, half].wait()
                comm_ref[8 + q_idx] = comm_ref[q_idx] + comm_ref[4 + q_idx]
                r2[b, half] = xchg(8 + q_idx, 12 + q_idx, 4 + q_idx,
                                   r2_partner[half])
                r2[b, half].start()

        for b in range(B):
            for half in range(2):
                q_idx = b * 2 + half
                out_ref[b, :, half * HALF:(half + 1) * HALF] += (
                    comm_ref[4 + q_idx].astype(jnp.float32))

        for b in range(B):
            for half in range(2):
                q_idx = b * 2 + half
                r2[b, half].wait()
                out_ref[b, :, half * HALF:(half + 1) * HALF] += (
                    comm_ref[12 + q_idx].astype(jnp.float32))

    return pl.pallas_call(
        body,
        out_shape=jax.ShapeDtypeStruct((B, SQ, D_MODEL), jnp.float32),
        in_specs=[pl.BlockSpec(memory_space=pltpu.VMEM)] * 5,
        out_specs=pl.BlockSpec(memory_space=pltpu.VMEM),
        scratch_shapes=[
            pltpu.VMEM((16, SQ, HALF), jnp.bfloat16),
            pltpu.SemaphoreType.DMA((8,)),
            pltpu.SemaphoreType.DMA((8,)),
        ],
        compiler_params=pltpu.CompilerParams(collective_id=0),
    )(x, Wq, K_loc, V_loc, Wo)
